# baseline (speedup 1.0000x reference)
import functools

import jax
import jax.numpy as jnp
from jax import lax
from jax.experimental import pallas as pl
from jax.experimental.pallas import tpu as pltpu
from jax.experimental.pallas import tpu_sc as plsc

HASH_SIZE = 2 ** 22
CELL_SIZE = 0.001
DIM = 4
PACK = 16 // DIM

_M1 = 2246822507
_M2 = 3266489909
_M1L, _M1H = _M1 & 0xFFFF, _M1 >> 16
_M2L, _M2H = _M2 & 0xFFFF, _M2 >> 16

_NC = 2
_NS = 16
_NW = _NC * _NS
_L = 16

_CHUNK = 2048
_STREAM = 128


def _i32(v):
    return jnp.int32(v)


def _hash16(ix, iy):
    mask16 = _i32(0xFFFF)
    c13 = _i32(13)
    c16 = _i32(16)
    c3 = _i32(3)
    a = ix * _i32(_M1L)
    b = ix * _i32(_M1H)
    l0 = a & mask16
    s1 = (a >> c16) + (b & mask16)
    l1 = s1 & mask16
    l2 = (b >> c16) + (s1 >> c16)
    q0 = l0 ^ (((l0 >> c13) | (l1 << c3)) & mask16)
    q1 = l1 ^ (((l1 >> c13) | (l2 << c3)) & mask16)
    c = iy * _i32(_M2L)
    d = iy * _i32(_M2H)
    g0 = c & mask16
    t1 = (c >> c16) + (d & mask16)
    g1 = t1 & mask16
    g2 = (d >> c16) + (t1 >> c16)
    u0 = q0 + g0
    r0 = u0 & mask16
    u1 = q1 + g1 + (u0 >> c16)
    r1 = u1 & mask16
    r2 = l2 + g2 + (u1 >> c16)
    s0 = r0 ^ r1
    s1b = r1 ^ r2
    return s0 | ((s1b & _i32(0x3F)) << c16)


def _make_kernel(n):
    per_w = n // _NW
    n_chunks = per_w // _CHUNK
    n_streams = _CHUNK // _STREAM
    mesh = plsc.VectorSubcoreMesh(
        core_axis_name="c", subcore_axis_name="s",
        num_cores=_NC, num_subcores=_NS)

    @functools.partial(
        pl.kernel,
        mesh=mesh,
        out_type=jax.ShapeDtypeStruct((n, DIM), jnp.float32),
        scratch_types=[
            pltpu.VMEM((_CHUNK * 2,), jnp.float32),
            pltpu.VMEM((n_streams, _STREAM), jnp.int32),
            pltpu.VMEM((_CHUNK,), jnp.int32),
            pltpu.VMEM((_CHUNK, 16), jnp.float32),
            pltpu.VMEM((_CHUNK, DIM), jnp.float32),
            pltpu.SemaphoreType.DMA,
        ],
        compiler_params=pltpu.CompilerParams(
            needs_layout_passes=False, use_tc_tiling_on_sc=False),
    )
    def k(pos_hbm, grid_hbm, out_hbm, pos_v, idx_v, sub_v, raw_v, out_v, sem):
        wid = lax.axis_index("s") * _i32(_NC) + lax.axis_index("c")
        base = wid * _i32(per_w)
        lanes = lax.iota(jnp.int32, _L)
        lanes2 = lanes * _i32(2)

        def chunk_body(ci, carry):
            off = base + ci * _i32(_CHUNK)
            pltpu.sync_copy(
                pos_hbm.at[pl.ds(off * _i32(2), _CHUNK * 2)], pos_v)

            def vec_body(kk, carry2):
                xi = kk * _i32(2 * _L) + lanes2
                x = plsc.load_gather(pos_v, [xi])
                y = plsc.load_gather(pos_v, [xi + _i32(1)])
                ix = (x / CELL_SIZE).astype(jnp.int32)
                iy = (y / CELL_SIZE).astype(jnp.int32)
                idx = _hash16(ix, iy)
                r = kk >> _i32(3)
                c0 = (kk & _i32(7)) * _i32(_L)
                idx_v[r, pl.ds(c0, _L)] = idx >> _i32(2)
                sub_v[pl.ds(kk * _i32(_L), _L)] = (idx & _i32(3)) * _i32(DIM)
                return carry2

            lax.fori_loop(_i32(0), _i32(_CHUNK // _L), vec_body, _i32(0))

            copies = [
                pltpu.async_copy(
                    grid_hbm.at[idx_v.at[_i32(j)]],
                    raw_v.at[pl.ds(j * _STREAM, _STREAM)],
                    sem,
                )
                for j in range(n_streams)
            ]
            for cp in copies:
                cp.wait()

            def sel_body(kk, carry2):
                o = kk * _i32(_L)
                rows16 = o + lanes
                s = sub_v[pl.ds(o, _L)]
                for dcomp in range(DIM):
                    v = plsc.load_gather(raw_v, [rows16, s + _i32(dcomp)])
                    plsc.store_scatter(
                        out_v, [rows16, lanes * _i32(0) + _i32(dcomp)], v)
                return carry2

            lax.fori_loop(_i32(0), _i32(_CHUNK // _L), sel_body, _i32(0))
            pltpu.sync_copy(out_v, out_hbm.at[pl.ds(off, _CHUNK)])
            return carry

        lax.fori_loop(_i32(0), _i32(n_chunks), chunk_body, _i32(0))

    return k


def kernel(positions, grid):
    n = positions.shape[0]
    grid16 = grid.reshape(HASH_SIZE // PACK, DIM * PACK)
    return _make_kernel(n)(positions.reshape(-1), grid16)


# speedup vs baseline: 30.4605x; 30.4605x over previous
import functools

import jax
import jax.numpy as jnp
from jax import lax
from jax.experimental import pallas as pl
from jax.experimental.pallas import tpu as pltpu
from jax.experimental.pallas import tpu_sc as plsc

HASH_SIZE = 2 ** 22
CELL_SIZE = 0.001
DIM = 4

_M1 = 2246822507
_M2 = 3266489909
_M1L, _M1H = _M1 & 0xFFFF, _M1 >> 16
_M2L, _M2H = _M2 & 0xFFFF, _M2 >> 16

_NC = 2
_NS = 16
_NW = _NC * _NS
_L = 16

_CHUNK = 2048
_STREAM = 128


def _i32(v):
    return jnp.int32(v)


def _hash16(ix, iy):
    mask16 = _i32(0xFFFF)
    c13 = _i32(13)
    c16 = _i32(16)
    c3 = _i32(3)
    a = ix * _i32(_M1L)
    b = ix * _i32(_M1H)
    l0 = a & mask16
    s1 = (a >> c16) + (b & mask16)
    l1 = s1 & mask16
    l2 = (b >> c16) + (s1 >> c16)
    q0 = l0 ^ (((l0 >> c13) | (l1 << c3)) & mask16)
    q1 = l1 ^ (((l1 >> c13) | (l2 << c3)) & mask16)
    c = iy * _i32(_M2L)
    d = iy * _i32(_M2H)
    g0 = c & mask16
    t1 = (c >> c16) + (d & mask16)
    g1 = t1 & mask16
    g2 = (d >> c16) + (t1 >> c16)
    u0 = q0 + g0
    r0 = u0 & mask16
    u1 = q1 + g1 + (u0 >> c16)
    r1 = u1 & mask16
    r2 = l2 + g2 + (u1 >> c16)
    s0 = r0 ^ r1
    s1b = r1 ^ r2
    return s0 | ((s1b & _i32(0x3F)) << c16)


def _make_kernel(n):
    per_w = n // _NW
    n_chunks = per_w // _CHUNK
    n_streams = _CHUNK * DIM // _STREAM
    mesh = plsc.VectorSubcoreMesh(
        core_axis_name="c", subcore_axis_name="s",
        num_cores=_NC, num_subcores=_NS)

    @functools.partial(
        pl.kernel,
        mesh=mesh,
        out_type=jax.ShapeDtypeStruct((n * DIM,), jnp.float32),
        scratch_types=[
            pltpu.VMEM((_CHUNK * 2,), jnp.float32),
            pltpu.VMEM((n_streams, _STREAM), jnp.int32),
            pltpu.VMEM((_CHUNK * DIM,), jnp.float32),
            pltpu.SemaphoreType.DMA,
        ],
        compiler_params=pltpu.CompilerParams(
            needs_layout_passes=False, use_tc_tiling_on_sc=False),
    )
    def k(pos_hbm, grid_hbm, out_hbm, pos_v, idx_v, stage_v, sem):
        wid = lax.axis_index("s") * _i32(_NC) + lax.axis_index("c")
        base = wid * _i32(per_w)
        lanes = lax.iota(jnp.int32, _L)

        def chunk_body(ci, carry):
            off = base + ci * _i32(_CHUNK)
            pltpu.sync_copy(
                pos_hbm.at[pl.ds(off * _i32(2), _CHUNK * 2)], pos_v)

            def vec_body(kk, carry2):
                blk = kk >> _i32(3)
                r0 = (kk & _i32(7)) * _i32(_L)
                po = blk * _i32(256) + r0
                x = pos_v[pl.ds(po, _L)]
                y = pos_v[pl.ds(po + _i32(128), _L)]
                ix = (x / CELL_SIZE).astype(jnp.int32)
                iy = (y / CELL_SIZE).astype(jnp.int32)
                idx = _hash16(ix, iy)
                w0 = ((idx >> _i32(7)) << _i32(9)) + (idx & _i32(127))
                row = blk * _i32(DIM)
                idx_v[row, pl.ds(r0, _L)] = w0
                idx_v[row + _i32(1), pl.ds(r0, _L)] = w0 + _i32(128)
                idx_v[row + _i32(2), pl.ds(r0, _L)] = w0 + _i32(256)
                idx_v[row + _i32(3), pl.ds(r0, _L)] = w0 + _i32(384)
                return carry2

            lax.fori_loop(_i32(0), _i32(_CHUNK // _L), vec_body, _i32(0))

            copies = [
                pltpu.async_copy(
                    grid_hbm.at[idx_v.at[_i32(j)]],
                    stage_v.at[pl.ds(j * _STREAM, _STREAM)],
                    sem,
                )
                for j in range(n_streams)
            ]
            for cp in copies:
                cp.wait()
            pltpu.sync_copy(
                stage_v, out_hbm.at[pl.ds(off * _i32(DIM), _CHUNK * DIM)])
            return carry

        lax.fori_loop(_i32(0), _i32(n_chunks), chunk_body, _i32(0))

    return k


def kernel(positions, grid):
    n = positions.shape[0]
    posv = positions.reshape(n // 128, 128, 2).transpose(0, 2, 1).reshape(-1)
    gridv = grid.reshape(HASH_SIZE // 128, 128, DIM)
    gridv = gridv.transpose(0, 2, 1).reshape(-1)
    out1d = _make_kernel(n)(posv, gridv)
    return out1d.reshape(n // 128, DIM, 128).transpose(0, 2, 1).reshape(n, DIM)
